# baseline (speedup 1.0000x reference)
import numpy as np
import jax
import jax.numpy as jnp
from jax.experimental import pallas as pl
from jax.experimental.pallas import tpu as pltpu

_MASKING_RATE = 0.4
_MSK_SCALAR = 0.0


def _fused_kernel(n_mask, r_ref, x1_ref, x2_ref, x3_ref,
                  o1_ref, o2_ref, o3_ref, m_ref, mscr_ref):
    ti = pl.program_id(0)
    bi = pl.program_id(1)
    t = r_ref.shape[1]
    tb = x1_ref.shape[1]

    @pl.when(bi == 0)
    def _():
        r = r_ref[0, :]
        rows = r_ref[0, pl.ds(ti * tb, tb)]
        rj = r[None, :]
        rt = rows[:, None]
        jidx = jax.lax.broadcasted_iota(jnp.int32, (tb, t), 1)
        tidx = ti * tb + jax.lax.broadcasted_iota(jnp.int32, (tb, t), 0)
        before = (rj < rt) | ((rj == rt) & (jidx < tidx))
        ranks = jnp.sum(before.astype(jnp.int32), axis=1, keepdims=True)
        mcol = (ranks < n_mask).astype(jnp.float32)
        mscr_ref[...] = mcol
        m_ref[0, :] = mcol[:, 0]

    masked = mscr_ref[...] != 0.0
    o1_ref[0] = jnp.where(masked, _MSK_SCALAR, x1_ref[0])
    o2_ref[0] = jnp.where(masked, _MSK_SCALAR, x2_ref[0])
    o3_ref[0] = jnp.where(masked, _MSK_SCALAR, x3_ref[0])


def kernel(x_tre, x_sea, x_res):
    b, t, f = x_tre.shape
    n_mask = int(np.ceil(t * _MASKING_RATE))
    rk = jax.random.key(42)
    r = jax.random.uniform(rk, (t,), minval=0.0, maxval=1.0)

    tb = 512
    x_spec = pl.BlockSpec((1, tb, f), lambda ti, bi: (bi, ti, 0))
    r_spec = pl.BlockSpec((1, t), lambda ti, bi: (0, 0))
    m_spec = pl.BlockSpec((1, tb), lambda ti, bi: (0, ti))
    z_tre, z_sea, z_res, mask = pl.pallas_call(
        lambda *refs: _fused_kernel(n_mask, *refs),
        grid=(t // tb, b),
        in_specs=[r_spec, x_spec, x_spec, x_spec],
        out_specs=[x_spec, x_spec, x_spec, m_spec],
        out_shape=[jax.ShapeDtypeStruct((b, t, f), jnp.float32)] * 3
        + [jax.ShapeDtypeStruct((1, t), jnp.float32)],
        scratch_shapes=[pltpu.VMEM((tb, 1), jnp.float32)],
        compiler_params=pltpu.CompilerParams(
            dimension_semantics=("arbitrary", "arbitrary"),
        ),
    )(r[None, :], x_tre, x_sea, x_res)

    return (z_tre, z_sea, z_res, mask[0] != 0.0)


# speedup vs baseline: 1.0381x; 1.0381x over previous
import numpy as np
import jax
import jax.numpy as jnp
from jax.experimental import pallas as pl
from jax.experimental.pallas import tpu as pltpu

_MASKING_RATE = 0.4
_MSK_SCALAR = 0.0


def _fused_kernel(n_mask, r_ref, x1_ref, x2_ref, x3_ref,
                  o1_ref, o2_ref, o3_ref, m_ref, mscr_ref):
    ti = pl.program_id(0)
    bi = pl.program_id(1)
    t = r_ref.shape[1]
    tb = x1_ref.shape[1]

    @pl.when(bi == 0)
    def _():
        r = r_ref[0, :]
        rows = r_ref[0, pl.ds(ti * tb, tb)]
        rj = r[None, :]
        rt = rows[:, None]
        jidx = jax.lax.broadcasted_iota(jnp.int32, (tb, t), 1)
        tidx = ti * tb + jax.lax.broadcasted_iota(jnp.int32, (tb, t), 0)
        before = (rj < rt) | ((rj == rt) & (jidx < tidx))
        ranks = jnp.sum(before.astype(jnp.int32), axis=1, keepdims=True)
        mcol = (ranks < n_mask).astype(jnp.float32)
        mscr_ref[...] = mcol
        m_ref[0, :] = mcol[:, 0]

    masked = mscr_ref[...] != 0.0
    o1_ref[0] = jnp.where(masked, _MSK_SCALAR, x1_ref[0])
    o2_ref[0] = jnp.where(masked, _MSK_SCALAR, x2_ref[0])
    o3_ref[0] = jnp.where(masked, _MSK_SCALAR, x3_ref[0])


def kernel(x_tre, x_sea, x_res):
    b, t, f = x_tre.shape
    n_mask = int(np.ceil(t * _MASKING_RATE))
    rk = jax.random.key(42)
    r = jax.random.uniform(rk, (t,), minval=0.0, maxval=1.0)

    tb = 1024
    x_spec = pl.BlockSpec((1, tb, f), lambda ti, bi: (bi, ti, 0))
    r_spec = pl.BlockSpec((1, t), lambda ti, bi: (0, 0))
    m_spec = pl.BlockSpec((1, tb), lambda ti, bi: (0, ti))
    z_tre, z_sea, z_res, mask = pl.pallas_call(
        lambda *refs: _fused_kernel(n_mask, *refs),
        grid=(t // tb, b),
        in_specs=[r_spec, x_spec, x_spec, x_spec],
        out_specs=[x_spec, x_spec, x_spec, m_spec],
        out_shape=[jax.ShapeDtypeStruct((b, t, f), jnp.float32)] * 3
        + [jax.ShapeDtypeStruct((1, t), jnp.float32)],
        scratch_shapes=[pltpu.VMEM((tb, 1), jnp.float32)],
        compiler_params=pltpu.CompilerParams(
            dimension_semantics=("arbitrary", "arbitrary"),
        ),
    )(r[None, :], x_tre, x_sea, x_res)

    return (z_tre, z_sea, z_res, mask[0] != 0.0)
